# baseline (speedup 1.0000x reference)
import functools

import jax
import jax.numpy as jnp
from jax.experimental import pallas as pl
from jax.experimental.pallas import tpu as pltpu

F = 128
BR1 = 320
BR = 640
_PAD = 640
ASCALE = 8192.0
LOSCALE = 16.0
F8 = jnp.float8_e4m3fn


def _rows_lt(n, base_rows, shape):
    rows = base_rows + jax.lax.broadcasted_iota(jnp.int32, shape, 0)
    return rows < n


def _blockmax(s):
    return jnp.max(jnp.abs(s), axis=(0, 1), keepdims=True).reshape(1, 1, 1)


def _next_scale(pm_ref, grow_ref):
    m = jnp.max(pm_ref[...], axis=(0, 2), keepdims=True).reshape(1, 1)
    g = grow_ref[...]
    bound = jnp.maximum(m * g[:, 0:1] + g[:, 1:2], 1e-30)
    return jnp.exp2(jnp.floor(jnp.log2(256.0 / bound)))


def _dot_mimic(h, w_ref):
    s = jnp.dot(h.astype(jnp.bfloat16), w_ref[...],
                preferred_element_type=jnp.float32)
    return s.astype(jnp.bfloat16).astype(jnp.float32)


def _quant_hilo(sm, sc):
    s = sm * sc
    hi = s.astype(F8)
    r1 = s - hi.astype(jnp.float32)
    mi = (r1 * LOSCALE).astype(F8)
    lo = ((r1 - mi.astype(jnp.float32) * (1.0 / LOSCALE))
          * (LOSCALE * LOSCALE)).astype(F8)
    return hi, mi, lo


def _acc_fp8(aq_ref, hi_ref, mi_ref, lo_ref, sinv_ref, crow_ref):
    acc_hi = jnp.dot(aq_ref[...], hi_ref[...],
                     preferred_element_type=jnp.float32)
    acc_mi = jnp.dot(aq_ref[...], mi_ref[...],
                     preferred_element_type=jnp.float32)
    acc_lo = jnp.dot(aq_ref[...], lo_ref[...],
                     preferred_element_type=jnp.float32)
    mult = sinv_ref[...].reshape(1, 1) * crow_ref[...]
    return (acc_hi + acc_mi * (1.0 / LOSCALE)
            + acc_lo * (1.0 / (LOSCALE * LOSCALE))) * mult


def _support0_body(n, x_ref, w_ref, o_ref, opm_ref):
    r = pl.program_id(0)
    s = jnp.dot(x_ref[...].astype(jnp.bfloat16), w_ref[...],
                preferred_element_type=jnp.float32)
    s = jnp.where(_rows_lt(n, r * o_ref.shape[0], s.shape), s, 0.0)
    o_ref[...] = s.astype(jnp.bfloat16)
    opm_ref[...] = _blockmax(s)


def _layer1_body(n, adj_ref, s_ref, w_ref, b_ref, pm_ref,
                 grow_ref, ones_ref, aq_ref, ohi_ref, omi_ref, olo_ref,
                 osinv_ref, opm_ref, ocrow_ref):
    r = pl.program_id(0)
    a = adj_ref[...]
    rows = _rows_lt(n, r * BR1, a.shape)
    cols = jax.lax.broadcasted_iota(jnp.int32, a.shape, 1) < n
    am = jnp.where(rows & cols, a, 0.0)
    aq = (am * ASCALE).astype(F8)
    aq_ref[...] = aq
    rs = jnp.dot(aq, ones_ref[...], preferred_element_type=jnp.float32
                 )[:, 0:1]
    ocrow_ref[...] = jnp.where(rs > 0.0, 1.0 / rs, 0.0)
    acc = jnp.dot(am.astype(jnp.bfloat16), s_ref[...],
                  preferred_element_type=jnp.float32)
    h = jnp.maximum(acc + b_ref[...], 0.0)
    s2 = _dot_mimic(h, w_ref)
    sm = jnp.where(_rows_lt(n, r * BR1, s2.shape), s2, 0.0)
    sc = _next_scale(pm_ref, grow_ref)
    hi, mi, lo = _quant_hilo(sm, sc)
    ohi_ref[...] = hi
    omi_ref[...] = mi
    olo_ref[...] = lo
    osinv_ref[...] = (1.0 / sc).reshape(1, 1, 1)
    opm_ref[...] = _blockmax(sm)


def _mid_body(n, aq_ref, hi_ref, mi_ref, lo_ref, sinv_ref, crow_ref,
              w_ref, b_ref, pm_ref, grow_ref,
              ohi_ref, omi_ref, olo_ref, osinv_ref, opm_ref):
    r = pl.program_id(0)
    acc = _acc_fp8(aq_ref, hi_ref, mi_ref, lo_ref, sinv_ref, crow_ref)
    h = jnp.maximum(acc + b_ref[...], 0.0)
    s2 = _dot_mimic(h, w_ref)
    sm = jnp.where(_rows_lt(n, r * BR, s2.shape), s2, 0.0)
    sc = _next_scale(pm_ref, grow_ref)
    hi, mi, lo = _quant_hilo(sm, sc)
    ohi_ref[...] = hi
    omi_ref[...] = mi
    olo_ref[...] = lo
    osinv_ref[...] = (1.0 / sc).reshape(1, 1, 1)
    opm_ref[...] = _blockmax(sm)


def _last_body(aq_ref, hi_ref, mi_ref, lo_ref, sinv_ref, crow_ref, b_ref,
               o_ref):
    o_ref[...] = _acc_fp8(aq_ref, hi_ref, mi_ref, lo_ref, sinv_ref,
                          crow_ref) + b_ref[...]


def _cparams():
    return pltpu.CompilerParams(dimension_semantics=("arbitrary",))


def kernel(x, adj, W1, b1, W2, b2, W3, b3, W4, b4, W5, b5, W6, b6):
    n = x.shape[0]
    np_ = ((n + _PAD - 1) // _PAD) * _PAD
    f32 = jnp.float32
    bf16 = jnp.bfloat16
    gr1, gr = np_ // BR1, np_ // BR
    w1b = W1.astype(bf16)
    w16 = [w.astype(bf16) for w in (W2, W3, W4, W5, W6)]
    bs = [b.reshape(1, F) for b in (b1, b2, b3, b4, b5, b6)]
    grow = []
    for w, b in ((W2, b1), (W3, b2), (W4, b3), (W5, b4), (W6, b5)):
        c = jnp.max(jnp.sum(jnp.abs(w), axis=0))
        grow.append(jnp.stack([c, c * jnp.max(jnp.abs(b))]).reshape(1, 2))

    full_s = pl.BlockSpec((np_, F), lambda r: (0, 0))
    full_w = pl.BlockSpec((F, F), lambda r: (0, 0))
    full_b = pl.BlockSpec((1, F), lambda r: (0, 0))
    full_g = pl.BlockSpec((1, 2), lambda r: (0, 0))
    pm_gr = pl.BlockSpec((gr, 1, 1), lambda r: (0, 0, 0))
    pm_gr1 = pl.BlockSpec((gr1, 1, 1), lambda r: (0, 0, 0))
    blk_pm = pl.BlockSpec((1, 1, 1), lambda r: (r, 0, 0))
    sinv_full = pl.BlockSpec((1, 1, 1), lambda r: (0, 0, 0))
    s_shape = jax.ShapeDtypeStruct((np_, F), F8)
    sinv_shape = jax.ShapeDtypeStruct((1, 1, 1), f32)

    s1, pm = pl.pallas_call(
        functools.partial(_support0_body, n),
        grid=(gr,),
        in_specs=[pl.BlockSpec((BR, F), lambda r: (r, 0)), full_w],
        out_specs=[pl.BlockSpec((BR, F), lambda r: (r, 0)), blk_pm],
        out_shape=[jax.ShapeDtypeStruct((np_, F), bf16),
                   jax.ShapeDtypeStruct((gr, 1, 1), f32)],
        compiler_params=_cparams(),
    )(x, w1b)

    ones8 = jnp.ones((np_, F), F8)
    blk_s = pl.BlockSpec((BR1, F), lambda r: (r, 0))
    adjq, shi, smi, slo, sinv, pm, crow = pl.pallas_call(
        functools.partial(_layer1_body, n),
        grid=(gr1,),
        in_specs=[pl.BlockSpec((BR1, np_), lambda r: (r, 0)),
                  full_s, full_w, full_b, pm_gr, full_g, full_s],
        out_specs=[pl.BlockSpec((BR1, np_), lambda r: (r, 0)),
                   blk_s, blk_s, blk_s, sinv_full, blk_pm,
                   pl.BlockSpec((BR1, 1), lambda r: (r, 0))],
        out_shape=[jax.ShapeDtypeStruct((np_, np_), F8),
                   s_shape, s_shape, s_shape,
                   sinv_shape, jax.ShapeDtypeStruct((gr1, 1, 1), f32),
                   jax.ShapeDtypeStruct((np_, 1), f32)],
        compiler_params=_cparams(),
    )(adj, s1, w16[0], bs[0], pm, grow[0], ones8)

    crow_spec = pl.BlockSpec((BR, 1), lambda r: (r, 0))
    blk_sm = pl.BlockSpec((BR, F), lambda r: (r, 0))
    for li in (1, 2, 3, 4):
        shi, smi, slo, sinv, pm = pl.pallas_call(
            functools.partial(_mid_body, n),
            grid=(gr,),
            in_specs=[pl.BlockSpec((BR, np_), lambda r: (r, 0)),
                      full_s, full_s, full_s, sinv_full, crow_spec,
                      full_w, full_b, pm_gr1 if li == 1 else pm_gr,
                      full_g],
            out_specs=[blk_sm, blk_sm, blk_sm, sinv_full, blk_pm],
            out_shape=[s_shape, s_shape, s_shape, sinv_shape,
                       jax.ShapeDtypeStruct((gr, 1, 1), f32)],
            compiler_params=_cparams(),
        )(adjq, shi, smi, slo, sinv, crow, w16[li], bs[li], pm, grow[li])

    out = pl.pallas_call(
        _last_body,
        grid=(gr,),
        in_specs=[pl.BlockSpec((BR, np_), lambda r: (r, 0)),
                  full_s, full_s, full_s, sinv_full, crow_spec, full_b],
        out_specs=pl.BlockSpec((BR, F), lambda r: (r, 0)),
        out_shape=jax.ShapeDtypeStruct((np_, F), f32),
        compiler_params=_cparams(),
    )(adjq, shi, smi, slo, sinv, crow, bs[5])

    return out[:n]


# speedup vs baseline: 1.3307x; 1.3307x over previous
import functools

import jax
import jax.numpy as jnp
from jax.experimental import pallas as pl
from jax.experimental.pallas import tpu as pltpu

F = 128
BR1 = 320
BR = 640
_PAD = 640
ASCALE = 8192.0
F8 = jnp.float8_e4m3fn


def _rows_lt(n, base_rows, shape):
    rows = base_rows + jax.lax.broadcasted_iota(jnp.int32, shape, 0)
    return rows < n


def _support0_body(n, x_ref, w_ref, o_ref):
    r = pl.program_id(0)
    s = jnp.dot(x_ref[...].astype(jnp.bfloat16), w_ref[...],
                preferred_element_type=jnp.float32)
    s = jnp.where(_rows_lt(n, r * o_ref.shape[0], s.shape), s, 0.0)
    o_ref[...] = s.astype(jnp.bfloat16)


def _epilogue(acc, b_ref, w_ref, rowmask):
    h = jnp.maximum(acc + b_ref[...], 0.0)
    s2 = jnp.dot(h.astype(jnp.bfloat16), w_ref[...],
                 preferred_element_type=jnp.float32)
    return jnp.where(rowmask, s2, 0.0).astype(jnp.bfloat16)


def _layer1_body(n, adj_ref, s_ref, w_ref, b_ref, ones_ref,
                 aq_ref, o_ref, ocrow_ref):
    r = pl.program_id(0)
    a = adj_ref[...]
    rows = _rows_lt(n, r * BR1, a.shape)
    cols = jax.lax.broadcasted_iota(jnp.int32, a.shape, 1) < n
    am = jnp.where(rows & cols, a, 0.0)
    aq = (am * ASCALE).astype(F8)
    aq_ref[...] = aq
    a16 = aq.astype(jnp.bfloat16)
    rs = jnp.dot(a16, ones_ref[...], preferred_element_type=jnp.float32
                 )[:, 0:1]
    ocrow_ref[...] = jnp.where(rs > 0.0, 1.0 / rs, 0.0)
    acc = jnp.dot(am.astype(jnp.bfloat16), s_ref[...],
                  preferred_element_type=jnp.float32)
    o_ref[...] = _epilogue(acc, b_ref, w_ref,
                           _rows_lt(n, r * BR1, (BR1, F)))


def _mid_body(n, aq_ref, s_ref, crow_ref, w_ref, b_ref, o_ref):
    r = pl.program_id(0)
    a16 = aq_ref[...].astype(jnp.bfloat16)
    acc = jnp.dot(a16, s_ref[...], preferred_element_type=jnp.float32)
    acc = acc * crow_ref[...]
    o_ref[...] = _epilogue(acc, b_ref, w_ref, _rows_lt(n, r * BR, (BR, F)))


def _last_body(aq_ref, s_ref, crow_ref, b_ref, o_ref):
    a16 = aq_ref[...].astype(jnp.bfloat16)
    acc = jnp.dot(a16, s_ref[...], preferred_element_type=jnp.float32)
    o_ref[...] = acc * crow_ref[...] + b_ref[...]


def _cparams():
    return pltpu.CompilerParams(dimension_semantics=("arbitrary",))


def kernel(x, adj, W1, b1, W2, b2, W3, b3, W4, b4, W5, b5, W6, b6):
    n = x.shape[0]
    np_ = ((n + _PAD - 1) // _PAD) * _PAD
    f32 = jnp.float32
    bf16 = jnp.bfloat16
    gr1, gr = np_ // BR1, np_ // BR
    w16 = [w.astype(bf16) for w in (W1, W2, W3, W4, W5, W6)]
    bs = [b.reshape(1, F) for b in (b1, b2, b3, b4, b5, b6)]

    full_s = pl.BlockSpec((np_, F), lambda r: (0, 0))
    full_w = pl.BlockSpec((F, F), lambda r: (0, 0))
    full_b = pl.BlockSpec((1, F), lambda r: (0, 0))

    s = pl.pallas_call(
        functools.partial(_support0_body, n),
        grid=(gr,),
        in_specs=[pl.BlockSpec((BR, F), lambda r: (r, 0)), full_w],
        out_specs=pl.BlockSpec((BR, F), lambda r: (r, 0)),
        out_shape=jax.ShapeDtypeStruct((np_, F), bf16),
        compiler_params=_cparams(),
    )(x, w16[0])

    ones16 = jnp.ones((np_, F), bf16)
    adjq, s, crow = pl.pallas_call(
        functools.partial(_layer1_body, n),
        grid=(gr1,),
        in_specs=[pl.BlockSpec((BR1, np_), lambda r: (r, 0)),
                  full_s, full_w, full_b, full_s],
        out_specs=[pl.BlockSpec((BR1, np_), lambda r: (r, 0)),
                   pl.BlockSpec((BR1, F), lambda r: (r, 0)),
                   pl.BlockSpec((BR1, 1), lambda r: (r, 0))],
        out_shape=[jax.ShapeDtypeStruct((np_, np_), F8),
                   jax.ShapeDtypeStruct((np_, F), bf16),
                   jax.ShapeDtypeStruct((np_, 1), f32)],
        compiler_params=_cparams(),
    )(adj, s, w16[1], bs[0], ones16)

    crow_spec = pl.BlockSpec((BR, 1), lambda r: (r, 0))
    for li in (1, 2, 3, 4):
        s = pl.pallas_call(
            functools.partial(_mid_body, n),
            grid=(gr,),
            in_specs=[pl.BlockSpec((BR, np_), lambda r: (r, 0)),
                      full_s, crow_spec, full_w, full_b],
            out_specs=pl.BlockSpec((BR, F), lambda r: (r, 0)),
            out_shape=jax.ShapeDtypeStruct((np_, F), bf16),
            compiler_params=_cparams(),
        )(adjq, s, crow, w16[li + 1], bs[li])

    out = pl.pallas_call(
        _last_body,
        grid=(gr,),
        in_specs=[pl.BlockSpec((BR, np_), lambda r: (r, 0)),
                  full_s, crow_spec, full_b],
        out_specs=pl.BlockSpec((BR, F), lambda r: (r, 0)),
        out_shape=jax.ShapeDtypeStruct((np_, F), f32),
        compiler_params=_cparams(),
    )(adjq, s, crow, bs[5])

    return out[:n]
